# baseline (speedup 1.0000x reference)
import functools
import math

import jax
import jax.numpy as jnp
from jax import lax
from jax.experimental import pallas as pl
from jax.experimental.pallas import tpu as pltpu
from jax.experimental.pallas import tpu_sc as plsc

LANES = 16
GROUP = 128


def _emb_call(n_total, n_groups, dim, scale):
    mesh = plsc.VectorSubcoreMesh(core_axis_name="c", subcore_axis_name="s")
    nw = 32

    @functools.partial(
        pl.kernel,
        mesh=mesh,
        out_type=jax.ShapeDtypeStruct((n_total, dim), jnp.float32),
        scratch_types=[
            pltpu.VMEM((n_groups, GROUP), jnp.int32),
            pltpu.VMEM((GROUP, dim), jnp.float32),
            pltpu.SemaphoreType.DMA,
        ],
        compiler_params=pltpu.CompilerParams(use_tc_tiling_on_sc=False),
    )
    def emb_kernel(idx_hbm, table_hbm, out_hbm, idx_v, rows_v, sem):
        nc = 2
        wid = lax.axis_index("s") * nc + lax.axis_index("c")
        pltpu.sync_copy(idx_hbm.at[wid], idx_v)
        out_base = wid * (n_groups * GROUP)

        def group_body(g, carry):
            pltpu.async_copy(table_hbm.at[idx_v.at[g]], rows_v, sem).wait()

            def row_body(r, c2):
                for j in range(dim // LANES):
                    sl = pl.ds(j * LANES, LANES)
                    rows_v[r, sl] = rows_v[r, sl] * scale
                return c2

            lax.fori_loop(0, GROUP, row_body, 0)
            pltpu.sync_copy(
                rows_v, out_hbm.at[pl.ds(out_base + g * GROUP, GROUP)]
            )
            return carry

        lax.fori_loop(0, n_groups, group_body, 0)

    return emb_kernel


def kernel(token_ids_batch, embeddings_table):
    b, l = token_ids_batch.shape
    v, d = embeddings_table.shape
    n_total = b * l
    nw = 32
    assert n_total % (nw * GROUP) == 0
    n_groups = n_total // (nw * GROUP)
    scale = math.sqrt(d)

    idx = token_ids_batch.astype(jnp.int32).reshape(nw, n_groups, GROUP)
    out = _emb_call(n_total, n_groups, d, scale)(idx, embeddings_table)
    return out.reshape(b, l, d)


# speedup vs baseline: 1.0710x; 1.0710x over previous
import functools
import math

import jax
import jax.numpy as jnp
from jax import lax
from jax.experimental import pallas as pl
from jax.experimental.pallas import tpu as pltpu
from jax.experimental.pallas import tpu_sc as plsc

LANES = 16
GROUP = 128
NW = 32


def _emb_call(n_total, n_groups, dim, scale):
    mesh = plsc.VectorSubcoreMesh(core_axis_name="c", subcore_axis_name="s")

    @functools.partial(
        pl.kernel,
        mesh=mesh,
        out_type=jax.ShapeDtypeStruct((n_total, dim), jnp.float32),
        scratch_types=[
            pltpu.VMEM((n_groups, GROUP), jnp.int32),
            pltpu.VMEM((GROUP, dim), jnp.float32),
            pltpu.VMEM((GROUP, dim), jnp.float32),
            pltpu.VMEM((GROUP, dim), jnp.float32),
            pltpu.VMEM((GROUP, dim), jnp.float32),
            pltpu.SemaphoreType.DMA,
            pltpu.SemaphoreType.DMA,
            pltpu.SemaphoreType.DMA,
            pltpu.SemaphoreType.DMA,
        ],
        compiler_params=pltpu.CompilerParams(use_tc_tiling_on_sc=False),
    )
    def emb_kernel(idx_hbm, table_hbm, out_hbm, idx_v,
                   in0, in1, st0, st1, sg0, sg1, ss0, ss1):
        nc = 2
        wid = lax.axis_index("s") * nc + lax.axis_index("c")
        pltpu.sync_copy(idx_hbm.at[wid], idx_v)
        out_base = wid * (n_groups * GROUP)
        in_bufs = (in0, in1)
        st_bufs = (st0, st1)
        g_sems = (sg0, sg1)
        s_sems = (ss0, ss1)

        def out_slice(g):
            return out_hbm.at[pl.ds(out_base + g * GROUP, GROUP)]

        pltpu.async_copy(table_hbm.at[idx_v.at[0]], in0, sg0)
        pltpu.async_copy(table_hbm.at[idx_v.at[1]], in1, sg1)

        def scale_group(src, dst):
            def row_body(r8, c):
                for k in range(8):
                    r = r8 * 8 + k
                    for j in range(dim // LANES):
                        sl = pl.ds(j * LANES, LANES)
                        dst[r, sl] = src[r, sl] * scale
                return c

            lax.fori_loop(0, GROUP // 8, row_body, 0)

        def outer(g2, carry):
            for p in range(2):
                g = g2 * 2 + p
                inb, stb = in_bufs[p], st_bufs[p]

                pltpu.make_async_copy(
                    table_hbm.at[idx_v.at[g]], inb, g_sems[p]
                ).wait()

                @pl.when(g2 >= 1)
                def _():
                    pltpu.make_async_copy(
                        stb, out_slice(g - 2), s_sems[p]
                    ).wait()

                scale_group(inb, stb)

                @pl.when(g2 < (n_groups // 2) - 1)
                def _():
                    pltpu.async_copy(
                        table_hbm.at[idx_v.at[g + 2]], inb, g_sems[p]
                    )

                pltpu.async_copy(stb, out_slice(g), s_sems[p])
            return carry

        lax.fori_loop(0, n_groups // 2, outer, 0)

        pltpu.make_async_copy(st0, out_slice(n_groups - 2), ss0).wait()
        pltpu.make_async_copy(st1, out_slice(n_groups - 1), ss1).wait()

    return emb_kernel


def kernel(token_ids_batch, embeddings_table):
    b, l = token_ids_batch.shape
    v, d = embeddings_table.shape
    n_total = b * l
    assert n_total % (NW * GROUP) == 0
    n_groups = n_total // (NW * GROUP)
    assert n_groups % 2 == 0
    scale = math.sqrt(d)

    idx = token_ids_batch.astype(jnp.int32).reshape(NW, n_groups, GROUP)
    out = _emb_call(n_total, n_groups, d, scale)(idx, embeddings_table)
    return out.reshape(b, l, d)
